# Initial kernel scaffold; baseline (speedup 1.0000x reference)
import jax
import jax.numpy as jnp
from jax.experimental import pallas as pl


def kernel(x, edge_index, edge_attr, scalar, params):
    raise NotImplementedError("write your pallas kernel here")


# speedup vs baseline: 2.6239x; 2.6239x over previous
import functools

import jax
import jax.numpy as jnp
from jax import lax
from jax.experimental import pallas as pl
from jax.experimental.pallas import tpu as pltpu
from jax.experimental.pallas import tpu_sc as plsc

_N = 10000
_E = 320000
_H = 64
_BE = 2560
_EGRID = _E // _BE

_NC, _NS = 2, 16
_NW = _NC * _NS
_EPW = _E // _NW
_C = 80
_NCH = _EPW // _C
_NPAD = 10240
_NPT = _NPAD // _NS

_INTERPRET = False


def _dot(a, b):
    return jax.lax.dot(a, b, preferred_element_type=jnp.float32)


def _node_enc_body(x_ref, w1, b1, w2, b2, w3, b3, gb, wsd,
                   h_ref, p2_ref):
    t = jnp.maximum(_dot(x_ref[...], w1[...]) + b1[...], 0.0)
    t = jnp.maximum(_dot(t, w2[...]) + b2[...], 0.0)
    t = _dot(t, w3[...]) + b3[...]
    h = gb[0:1, :] * t + gb[1:2, :]
    h_ref[...] = h
    p2_ref[...] = _dot(h, wsd[...])


def _node_enc(x, p, gb, wsd):
    return pl.pallas_call(
        _node_enc_body,
        out_shape=(jax.ShapeDtypeStruct((_N, _H), jnp.float32),
                   jax.ShapeDtypeStruct((_N, 2 * _H), jnp.float32)),
        interpret=_INTERPRET,
    )(x, p[0]["W"], p[0]["b"].reshape(1, -1),
      p[1]["W"], p[1]["b"].reshape(1, -1),
      p[2]["W"], p[2]["b"].reshape(1, -1), gb, wsd)


def _mlp3_body(x_ref, w1, b1, w2, b2, w3, b3, o_ref):
    t = jnp.maximum(_dot(x_ref[...], w1[...]) + b1[...], 0.0)
    t = jnp.maximum(_dot(t, w2[...]) + b2[...], 0.0)
    o_ref[...] = _dot(t, w3[...]) + b3[...]


def _edge_enc(ea, p):
    wspec = pl.BlockSpec((3, _H), lambda i: (0, 0))
    hspec = pl.BlockSpec((_H, _H), lambda i: (0, 0))
    bspec = pl.BlockSpec((1, _H), lambda i: (0, 0))
    return pl.pallas_call(
        _mlp3_body,
        grid=(_EGRID,),
        in_specs=[pl.BlockSpec((_BE, 3), lambda i: (i, 0)),
                  wspec, bspec, hspec, bspec, hspec, bspec],
        out_specs=pl.BlockSpec((_BE, _H), lambda i: (i, 0)),
        out_shape=jax.ShapeDtypeStruct((_E, _H), jnp.float32),
        interpret=_INTERPRET,
    )(ea, p[0]["W"], p[0]["b"].reshape(1, -1),
      p[1]["W"], p[1]["b"].reshape(1, -1),
      p[2]["W"], p[2]["b"].reshape(1, -1))


def _edge_mlp_body(g_ref, ea_ref, w1e, b1, w2, b2, w3, b3, o_ref):
    t = jnp.maximum(g_ref[...] + _dot(ea_ref[...], w1e[...]) + b1[...], 0.0)
    t = jnp.maximum(_dot(t, w2[...]) + b2[...], 0.0)
    o_ref[:, 0:_H] = _dot(t, w3[...]) + b3[...]
    o_ref[:, _H:2 * _H] = jnp.ones((_BE, _H), jnp.float32)


def _edge_mlp(g, ea, w1e, p):
    ew = ea.shape[1]
    if ew == 2 * _H:
        w1e = jnp.concatenate([w1e, jnp.zeros((_H, _H), jnp.float32)], 0)
    hspec = pl.BlockSpec((_H, _H), lambda i: (0, 0))
    bspec = pl.BlockSpec((1, _H), lambda i: (0, 0))
    espec = pl.BlockSpec((_BE, _H), lambda i: (i, 0))
    return pl.pallas_call(
        _edge_mlp_body,
        grid=(_EGRID,),
        in_specs=[espec, pl.BlockSpec((_BE, ew), lambda i: (i, 0)),
                  pl.BlockSpec((ew, _H), lambda i: (0, 0)), bspec,
                  hspec, bspec, hspec, bspec],
        out_specs=pl.BlockSpec((_BE, 2 * _H), lambda i: (i, 0)),
        out_shape=jax.ShapeDtypeStruct((_E, 2 * _H), jnp.float32),
        interpret=_INTERPRET,
    )(g, ea, w1e, p[0]["b"].reshape(1, -1),
      p[1]["W"], p[1]["b"].reshape(1, -1),
      p[2]["W"], p[2]["b"].reshape(1, -1))


def _agg_from_s(s_ref):
    cnt = s_ref[0, :, _H:_H + 1] + s_ref[1, :, _H:_H + 1]
    return (s_ref[0, :, 0:_H] + s_ref[1, :, 0:_H]) / jnp.maximum(cnt, 1.0)


def _node_mlp_body(h_ref, s_ref, w1h, w1a, b1, w2, b2, w3, b3,
                   wsd, h_out, p2_out):
    agg = _agg_from_s(s_ref)
    t = jnp.maximum(_dot(h_ref[...], w1h[...]) + _dot(agg, w1a[...])
                    + b1[...], 0.0)
    t = jnp.maximum(_dot(t, w2[...]) + b2[...], 0.0)
    h = _dot(t, w3[...]) + b3[...]
    h_out[...] = h
    p2_out[...] = _dot(h, wsd[...])


def _node_mlp(h, s, w1h, w1a, p, wsd):
    return pl.pallas_call(
        _node_mlp_body,
        out_shape=(jax.ShapeDtypeStruct((_N, _H), jnp.float32),
                   jax.ShapeDtypeStruct((_N, 2 * _H), jnp.float32)),
        interpret=_INTERPRET,
    )(h, s, w1h, w1a, p[0]["b"].reshape(1, -1),
      p[1]["W"], p[1]["b"].reshape(1, -1),
      p[2]["W"], p[2]["b"].reshape(1, -1), wsd)


def _node_last_body(h_ref, s_ref, w1h, w1a, b1, w2, b2, w3, b3,
                    d1, db1, d2, db2, d3, db3, o_ref):
    agg = _agg_from_s(s_ref)
    t = jnp.maximum(_dot(h_ref[...], w1h[...]) + _dot(agg, w1a[...])
                    + b1[...], 0.0)
    t = jnp.maximum(_dot(t, w2[...]) + b2[...], 0.0)
    h = _dot(t, w3[...]) + b3[...]
    t = jnp.maximum(_dot(h, d1[...]) + db1[...], 0.0)
    t = jnp.maximum(_dot(t, d2[...]) + db2[...], 0.0)
    o_ref[...] = _dot(t, d3[...]) + db3[...]


def _node_last(h, s, w1h, w1a, p, dec):
    return pl.pallas_call(
        _node_last_body,
        out_shape=jax.ShapeDtypeStruct((_N, 1), jnp.float32),
        interpret=_INTERPRET,
    )(h, s, w1h, w1a, p[0]["b"].reshape(1, -1),
      p[1]["W"], p[1]["b"].reshape(1, -1),
      p[2]["W"], p[2]["b"].reshape(1, -1),
      dec[0]["W"], dec[0]["b"].reshape(1, -1),
      dec[1]["W"], dec[1]["b"].reshape(1, -1),
      dec[2]["W"], dec[2]["b"].reshape(1, -1))


def _sc_mesh():
    return plsc.VectorSubcoreMesh(core_axis_name="c", subcore_axis_name="s",
                                  num_cores=_NC, num_subcores=_NS)


def _gather_body(p2_hbm, row_hbm, col_hbm, out_hbm,
                 ridx, cidx, abuf, bbuf, obuf, gsem, osem):
    wid = lax.axis_index("s") * _NC + lax.axis_index("c")
    base = wid * _EPW

    def issue(slot, chunk):
        off = base + chunk * _C
        pltpu.sync_copy(row_hbm.at[pl.ds(off, _C)], ridx.at[slot])
        pltpu.sync_copy(col_hbm.at[pl.ds(off, _C)], cidx.at[slot])
        pltpu.async_copy(p2_hbm.at[ridx.at[slot]], abuf.at[slot],
                         gsem.at[slot])
        pltpu.async_copy(p2_hbm.at[cidx.at[slot]], bbuf.at[slot],
                         gsem.at[slot])

    issue(0, 0)

    def step(i, carry):
        slot = lax.rem(i, 2)
        nslot = 1 - slot

        @pl.when(i + 1 < _NCH)
        def _():
            issue(nslot, i + 1)

        pltpu.make_async_copy(p2_hbm.at[ridx.at[slot]], abuf.at[slot],
                              gsem.at[slot]).wait()
        pltpu.make_async_copy(p2_hbm.at[cidx.at[slot]], bbuf.at[slot],
                              gsem.at[slot]).wait()

        @pl.when(i >= 2)
        def _():
            pltpu.make_async_copy(obuf.at[slot],
                                  out_hbm.at[pl.ds(base, _C)],
                                  osem.at[slot]).wait()

        def add_row(r, c2):
            for k in range(_H // 16):
                obuf[slot, r, pl.ds(k * 16, 16)] = (
                    abuf[slot, r, pl.ds(k * 16, 16)]
                    + bbuf[slot, r, pl.ds(_H + k * 16, 16)])
            return c2

        lax.fori_loop(0, _C, add_row, 0, unroll=4)
        pltpu.async_copy(obuf.at[slot],
                         out_hbm.at[pl.ds(base + i * _C, _C)], osem.at[slot])
        return carry

    lax.fori_loop(0, _NCH, step, 0)
    for slot in (0, 1):
        pltpu.make_async_copy(obuf.at[slot], out_hbm.at[pl.ds(base, _C)],
                              osem.at[slot]).wait()


def _sc_gather(p2, row, col):
    f = pl.kernel(
        _gather_body,
        out_type=jax.ShapeDtypeStruct((_E, _H), jnp.float32),
        mesh=_sc_mesh(),
        scratch_types=[
            pltpu.VMEM((2, _C), jnp.int32),
            pltpu.VMEM((2, _C), jnp.int32),
            pltpu.VMEM((2, _C, 2 * _H), jnp.float32),
            pltpu.VMEM((2, _C, 2 * _H), jnp.float32),
            pltpu.VMEM((2, _C, _H), jnp.float32),
            pltpu.SemaphoreType.DMA((2,)),
            pltpu.SemaphoreType.DMA((2,)),
        ],
    )
    return f(p2, row, col)


def _make_scatter_body(width):
    def body(vals_hbm, col_hbm, out_hbm, idxb, vbuf, table, sem):
        cid = lax.axis_index("c")
        sid = lax.axis_index("s")
        wid = sid * _NC + cid
        base = wid * _EPW
        zv = jnp.zeros((16,), jnp.float32)

        def zrow(r, c2):
            for k in range(width // 16):
                vbuf[r, pl.ds(k * 16, 16)] = zv
            return c2

        lax.fori_loop(0, _C, zrow, 0, unroll=4)

        def zcopy(j, c2):
            pltpu.sync_copy(vbuf,
                            table.at[pl.ds(sid * _NPT + j * _C, _C)])
            return c2

        lax.fori_loop(0, _NPT // _C, zcopy, 0)
        plsc.subcore_barrier()

        def step(i, c2):
            off = base + i * _C
            pltpu.sync_copy(col_hbm.at[pl.ds(off, _C)], idxb.at[0])
            pltpu.sync_copy(vals_hbm.at[pl.ds(off, _C)], vbuf)
            pltpu.sync_copy(vbuf, table.at[idxb.at[0]], add=True)
            return c2

        lax.fori_loop(0, _NCH, step, 0)
        plsc.subcore_barrier()
        pltpu.sync_copy(table.at[pl.ds(sid * _NPT, _NPT)],
                        out_hbm.at[cid, pl.ds(sid * _NPT, _NPT)])

    return body


def _sc_scatter(vals, col, width):
    f = pl.kernel(
        _make_scatter_body(width),
        out_type=jax.ShapeDtypeStruct((_NC, _NPAD, width), jnp.float32),
        mesh=_sc_mesh(),
        scratch_types=[
            pltpu.VMEM((1, _C), jnp.int32),
            pltpu.VMEM((_C, width), jnp.float32),
            pltpu.VMEM_SHARED((_NPAD, width), jnp.float32),
            pltpu.SemaphoreType.DMA,
        ],
    )
    return f(vals, col)[:, :_N, :]


def kernel(x, edge_index, edge_attr, scalar, params):
    row, col = edge_index[0], edge_index[1]
    gb = (scalar @ params["film"]["W"] + params["film"]["b"]).reshape(2, _H)

    blocks = params["blocks"]
    ew = [{"wsd": jnp.concatenate([b["edge"][0]["W"][0:_H],
                                   b["edge"][0]["W"][_H:2 * _H]], axis=1),
           "we": b["edge"][0]["W"][2 * _H:]} for b in blocks]
    nw = [{"wh": b["node"][0]["W"][0:_H], "wa": b["node"][0]["W"][_H:]}
          for b in blocks]

    h, p2 = _node_enc(x, params["node_enc"], gb, ew[0]["wsd"])
    ea = _edge_enc(edge_attr, params["edge_enc"])

    for i in range(len(blocks)):
        g = _sc_gather(p2, row, col)
        ea = _edge_mlp(g, ea, ew[i]["we"], blocks[i]["edge"])
        s = _sc_scatter(ea, col, 2 * _H)
        if i + 1 < len(blocks):
            h, p2 = _node_mlp(h, s, nw[i]["wh"], nw[i]["wa"],
                              blocks[i]["node"], ew[i + 1]["wsd"])
        else:
            out = _node_last(h, s, nw[i]["wh"], nw[i]["wa"],
                             blocks[i]["node"], params["node_dec"])
    return out[:, 0]


# speedup vs baseline: 3.4301x; 1.3072x over previous
import functools

import jax
import jax.numpy as jnp
from jax import lax
from jax.experimental import pallas as pl
from jax.experimental.pallas import tpu as pltpu
from jax.experimental.pallas import tpu_sc as plsc

_N = 10000
_E = 320000
_H = 64
_BE = 2560
_EGRID = _E // _BE

_NC, _NS = 2, 16
_NW = _NC * _NS
_EPW = _E // _NW
_C = 80
_NCH = _EPW // _C
_NPAD = 10240
_NPT = _NPAD // _NS

_INTERPRET = False


def _dot(a, b):
    return jax.lax.dot(a, b, preferred_element_type=jnp.float32)


def _node_enc_body(x_ref, w1, b1, w2, b2, w3, b3, gb, wsd,
                   h_ref, p2_ref):
    t = jnp.maximum(_dot(x_ref[...], w1[...]) + b1[...], 0.0)
    t = jnp.maximum(_dot(t, w2[...]) + b2[...], 0.0)
    t = _dot(t, w3[...]) + b3[...]
    h = gb[0:1, :] * t + gb[1:2, :]
    h_ref[...] = h
    p2_ref[...] = _dot(h, wsd[...])


def _node_enc(x, p, gb, wsd):
    return pl.pallas_call(
        _node_enc_body,
        out_shape=(jax.ShapeDtypeStruct((_N, _H), jnp.float32),
                   jax.ShapeDtypeStruct((_N, 2 * _H), jnp.float32)),
        interpret=_INTERPRET,
    )(x, p[0]["W"], p[0]["b"].reshape(1, -1),
      p[1]["W"], p[1]["b"].reshape(1, -1),
      p[2]["W"], p[2]["b"].reshape(1, -1), gb, wsd)


def _mlp3_body(x_ref, w1, b1, w2, b2, w3, b3, o_ref):
    t = jnp.maximum(_dot(x_ref[...], w1[...]) + b1[...], 0.0)
    t = jnp.maximum(_dot(t, w2[...]) + b2[...], 0.0)
    o_ref[...] = _dot(t, w3[...]) + b3[...]


def _edge_enc(ea, p):
    wspec = pl.BlockSpec((3, _H), lambda i: (0, 0))
    hspec = pl.BlockSpec((_H, _H), lambda i: (0, 0))
    bspec = pl.BlockSpec((1, _H), lambda i: (0, 0))
    return pl.pallas_call(
        _mlp3_body,
        grid=(_EGRID,),
        in_specs=[pl.BlockSpec((_BE, 3), lambda i: (i, 0)),
                  wspec, bspec, hspec, bspec, hspec, bspec],
        out_specs=pl.BlockSpec((_BE, _H), lambda i: (i, 0)),
        out_shape=jax.ShapeDtypeStruct((_E, _H), jnp.float32),
        interpret=_INTERPRET,
    )(ea, p[0]["W"], p[0]["b"].reshape(1, -1),
      p[1]["W"], p[1]["b"].reshape(1, -1),
      p[2]["W"], p[2]["b"].reshape(1, -1))


def _edge_mlp_body(g_ref, ea_ref, w1e, b1, w2, b2, w3, b3, o_ref):
    t = jnp.maximum(g_ref[...] + _dot(ea_ref[...], w1e[...]) + b1[...], 0.0)
    t = jnp.maximum(_dot(t, w2[...]) + b2[...], 0.0)
    o_ref[:, 0:_H] = _dot(t, w3[...]) + b3[...]
    o_ref[:, _H:2 * _H] = jnp.ones((_BE, _H), jnp.float32)


def _edge_mlp(g, ea, w1e, p):
    ew = ea.shape[1]
    if ew == 2 * _H:
        w1e = jnp.concatenate([w1e, jnp.zeros((_H, _H), jnp.float32)], 0)
    hspec = pl.BlockSpec((_H, _H), lambda i: (0, 0))
    bspec = pl.BlockSpec((1, _H), lambda i: (0, 0))
    espec = pl.BlockSpec((_BE, _H), lambda i: (i, 0))
    return pl.pallas_call(
        _edge_mlp_body,
        grid=(_EGRID,),
        in_specs=[espec, pl.BlockSpec((_BE, ew), lambda i: (i, 0)),
                  pl.BlockSpec((ew, _H), lambda i: (0, 0)), bspec,
                  hspec, bspec, hspec, bspec],
        out_specs=pl.BlockSpec((_BE, 2 * _H), lambda i: (i, 0)),
        out_shape=jax.ShapeDtypeStruct((_E, 2 * _H), jnp.float32),
        interpret=_INTERPRET,
    )(g, ea, w1e, p[0]["b"].reshape(1, -1),
      p[1]["W"], p[1]["b"].reshape(1, -1),
      p[2]["W"], p[2]["b"].reshape(1, -1))


def _agg_from_s(s_ref):
    cnt = s_ref[0, :, _H:_H + 1] + s_ref[1, :, _H:_H + 1]
    return (s_ref[0, :, 0:_H] + s_ref[1, :, 0:_H]) / jnp.maximum(cnt, 1.0)


def _node_mlp_body(h_ref, s_ref, w1h, w1a, b1, w2, b2, w3, b3,
                   wsd, h_out, p2_out):
    agg = _agg_from_s(s_ref)
    t = jnp.maximum(_dot(h_ref[...], w1h[...]) + _dot(agg, w1a[...])
                    + b1[...], 0.0)
    t = jnp.maximum(_dot(t, w2[...]) + b2[...], 0.0)
    h = _dot(t, w3[...]) + b3[...]
    h_out[...] = h
    p2_out[...] = _dot(h, wsd[...])


def _node_mlp(h, s, w1h, w1a, p, wsd):
    return pl.pallas_call(
        _node_mlp_body,
        out_shape=(jax.ShapeDtypeStruct((_N, _H), jnp.float32),
                   jax.ShapeDtypeStruct((_N, 2 * _H), jnp.float32)),
        interpret=_INTERPRET,
    )(h, s, w1h, w1a, p[0]["b"].reshape(1, -1),
      p[1]["W"], p[1]["b"].reshape(1, -1),
      p[2]["W"], p[2]["b"].reshape(1, -1), wsd)


def _node_last_body(h_ref, s_ref, w1h, w1a, b1, w2, b2, w3, b3,
                    d1, db1, d2, db2, d3, db3, o_ref):
    agg = _agg_from_s(s_ref)
    t = jnp.maximum(_dot(h_ref[...], w1h[...]) + _dot(agg, w1a[...])
                    + b1[...], 0.0)
    t = jnp.maximum(_dot(t, w2[...]) + b2[...], 0.0)
    h = _dot(t, w3[...]) + b3[...]
    t = jnp.maximum(_dot(h, d1[...]) + db1[...], 0.0)
    t = jnp.maximum(_dot(t, d2[...]) + db2[...], 0.0)
    o_ref[...] = _dot(t, d3[...]) + db3[...]


def _node_last(h, s, w1h, w1a, p, dec):
    return pl.pallas_call(
        _node_last_body,
        out_shape=jax.ShapeDtypeStruct((_N, 1), jnp.float32),
        interpret=_INTERPRET,
    )(h, s, w1h, w1a, p[0]["b"].reshape(1, -1),
      p[1]["W"], p[1]["b"].reshape(1, -1),
      p[2]["W"], p[2]["b"].reshape(1, -1),
      dec[0]["W"], dec[0]["b"].reshape(1, -1),
      dec[1]["W"], dec[1]["b"].reshape(1, -1),
      dec[2]["W"], dec[2]["b"].reshape(1, -1))


def _sc_mesh():
    return plsc.VectorSubcoreMesh(core_axis_name="c", subcore_axis_name="s",
                                  num_cores=_NC, num_subcores=_NS)


def _gather_body(p2_hbm, row_hbm, col_hbm, out_hbm,
                 ridx, cidx, abuf, bbuf, obuf, gsem, osem, isem):
    wid = lax.axis_index("s") * _NC + lax.axis_index("c")
    base = wid * _EPW

    def issue_idx(slot, chunk):
        off = base + chunk * _C
        pltpu.async_copy(row_hbm.at[pl.ds(off, _C)], ridx.at[slot],
                         isem.at[slot])
        pltpu.async_copy(col_hbm.at[pl.ds(off, _C)], cidx.at[slot],
                         isem.at[slot])

    def wait_idx(slot):
        for _ in range(2):
            pltpu.make_async_copy(row_hbm.at[pl.ds(base, _C)],
                                  ridx.at[slot], isem.at[slot]).wait()

    def issue_gather(slot):
        wait_idx(slot)
        pltpu.async_copy(p2_hbm.at[ridx.at[slot]], abuf.at[slot],
                         gsem.at[slot])
        pltpu.async_copy(p2_hbm.at[cidx.at[slot]], bbuf.at[slot],
                         gsem.at[slot])

    issue_idx(0, 0)
    issue_idx(1, 1)
    issue_gather(0)

    def step(i, carry):
        slot = lax.rem(i, 2)
        nslot = 1 - slot

        @pl.when(i + 1 < _NCH)
        def _():
            issue_gather(nslot)

        pltpu.make_async_copy(p2_hbm.at[ridx.at[slot]], abuf.at[slot],
                              gsem.at[slot]).wait()
        pltpu.make_async_copy(p2_hbm.at[cidx.at[slot]], bbuf.at[slot],
                              gsem.at[slot]).wait()

        @pl.when(i + 2 < _NCH)
        def _():
            issue_idx(slot, i + 2)

        @pl.when(i >= 2)
        def _():
            pltpu.make_async_copy(obuf.at[slot],
                                  out_hbm.at[pl.ds(base, _C)],
                                  osem.at[slot]).wait()

        def add_row(r, c2):
            for k in range(_H // 16):
                obuf[slot, r, pl.ds(k * 16, 16)] = (
                    abuf[slot, r, pl.ds(k * 16, 16)]
                    + bbuf[slot, r, pl.ds(_H + k * 16, 16)])
            return c2

        lax.fori_loop(0, _C, add_row, 0, unroll=4)
        pltpu.async_copy(obuf.at[slot],
                         out_hbm.at[pl.ds(base + i * _C, _C)], osem.at[slot])
        return carry

    lax.fori_loop(0, _NCH, step, 0)
    for slot in (0, 1):
        pltpu.make_async_copy(obuf.at[slot], out_hbm.at[pl.ds(base, _C)],
                              osem.at[slot]).wait()


def _sc_gather(p2, row, col):
    f = pl.kernel(
        _gather_body,
        out_type=jax.ShapeDtypeStruct((_E, _H), jnp.float32),
        mesh=_sc_mesh(),
        scratch_types=[
            pltpu.VMEM((2, _C), jnp.int32),
            pltpu.VMEM((2, _C), jnp.int32),
            pltpu.VMEM((2, _C, 2 * _H), jnp.float32),
            pltpu.VMEM((2, _C, 2 * _H), jnp.float32),
            pltpu.VMEM((2, _C, _H), jnp.float32),
            pltpu.SemaphoreType.DMA((2,)),
            pltpu.SemaphoreType.DMA((2,)),
            pltpu.SemaphoreType.DMA((2,)),
        ],
    )
    return f(p2, row, col)


def _make_scatter_body(width):
    def body(vals_hbm, col_hbm, out_hbm, idxb, vbuf, table,
             isem, vsem, asem):
        cid = lax.axis_index("c")
        sid = lax.axis_index("s")
        wid = sid * _NC + cid
        base = wid * _EPW
        zv = jnp.zeros((16,), jnp.float32)

        def zrow(r, c2):
            for k in range(width // 16):
                vbuf[0, r, pl.ds(k * 16, 16)] = zv
            return c2

        lax.fori_loop(0, _C, zrow, 0, unroll=4)

        def zcopy(j, c2):
            pltpu.sync_copy(vbuf.at[0],
                            table.at[pl.ds(sid * _NPT + j * _C, _C)])
            return c2

        lax.fori_loop(0, _NPT // _C, zcopy, 0)
        plsc.subcore_barrier()

        def issue_loads(slot, chunk):
            off = base + chunk * _C
            pltpu.async_copy(col_hbm.at[pl.ds(off, _C)], idxb.at[slot],
                             isem.at[slot])
            pltpu.async_copy(vals_hbm.at[pl.ds(off, _C)], vbuf.at[slot],
                             vsem.at[slot])

        def wait_add(slot):
            pltpu.make_async_copy(vbuf.at[slot], table.at[idxb.at[slot]],
                                  asem.at[slot]).wait()

        issue_loads(0, 0)

        def step(i, c2):
            slot = lax.rem(i, 2)
            nslot = 1 - slot
            pltpu.make_async_copy(col_hbm.at[pl.ds(base, _C)],
                                  idxb.at[slot], isem.at[slot]).wait()
            pltpu.make_async_copy(vals_hbm.at[pl.ds(base, _C)],
                                  vbuf.at[slot], vsem.at[slot]).wait()
            pltpu.async_copy(vbuf.at[slot], table.at[idxb.at[slot]],
                             asem.at[slot], add=True)

            @pl.when(i >= 1)
            def _():
                wait_add(nslot)

            @pl.when(i + 1 < _NCH)
            def _():
                issue_loads(nslot, i + 1)

            return c2

        lax.fori_loop(0, _NCH, step, 0)
        wait_add(0)
        plsc.subcore_barrier()
        pltpu.sync_copy(table.at[pl.ds(sid * _NPT, _NPT)],
                        out_hbm.at[cid, pl.ds(sid * _NPT, _NPT)])

    return body


def _sc_scatter(vals, col, width):
    f = pl.kernel(
        _make_scatter_body(width),
        out_type=jax.ShapeDtypeStruct((_NC, _NPAD, width), jnp.float32),
        mesh=_sc_mesh(),
        scratch_types=[
            pltpu.VMEM((2, _C), jnp.int32),
            pltpu.VMEM((2, _C, width), jnp.float32),
            pltpu.VMEM_SHARED((_NPAD, width), jnp.float32),
            pltpu.SemaphoreType.DMA((2,)),
            pltpu.SemaphoreType.DMA((2,)),
            pltpu.SemaphoreType.DMA((2,)),
        ],
    )
    return f(vals, col)[:, :_N, :]


def kernel(x, edge_index, edge_attr, scalar, params):
    row, col = edge_index[0], edge_index[1]
    gb = (scalar @ params["film"]["W"] + params["film"]["b"]).reshape(2, _H)

    blocks = params["blocks"]
    ew = [{"wsd": jnp.concatenate([b["edge"][0]["W"][0:_H],
                                   b["edge"][0]["W"][_H:2 * _H]], axis=1),
           "we": b["edge"][0]["W"][2 * _H:]} for b in blocks]
    nw = [{"wh": b["node"][0]["W"][0:_H], "wa": b["node"][0]["W"][_H:]}
          for b in blocks]

    h, p2 = _node_enc(x, params["node_enc"], gb, ew[0]["wsd"])
    ea = _edge_enc(edge_attr, params["edge_enc"])

    for i in range(len(blocks)):
        g = _sc_gather(p2, row, col)
        ea = _edge_mlp(g, ea, ew[i]["we"], blocks[i]["edge"])
        s = _sc_scatter(ea, col, 2 * _H)
        if i + 1 < len(blocks):
            h, p2 = _node_mlp(h, s, nw[i]["wh"], nw[i]["wa"],
                              blocks[i]["node"], ew[i + 1]["wsd"])
        else:
            out = _node_last(h, s, nw[i]["wh"], nw[i]["wa"],
                             blocks[i]["node"], params["node_dec"])
    return out[:, 0]
